# baseline (speedup 1.0000x reference)
import functools

import jax
import jax.numpy as jnp
from jax import lax
from jax.experimental import pallas as pl
from jax.experimental.pallas import tpu as pltpu
from jax.experimental.pallas import tpu_sc as plsc

_VOCAB = 100000
_D = 64
_MAX_NG = 16
_B = 16384
_NC = 2
_NS = 16
_NW = _NC * _NS
_BPW = _B // _NW
_NLANE = 16
_DV = _D // _NLANE
_FCH = 16
_CH = 16
_NCH = _BPW // _CH

_NGB = 500000 // 128
_INB = _VOCAB // 128


def _split(wid, total):
  per = total // _NW
  ext = total - per * _NW
  lo = wid * per + jnp.minimum(wid, ext)
  hi = lo + per + (wid < ext).astype(jnp.int32)
  return lo, hi


def _transpose_pack_phase(tab_t, pack_out, blk_v, outb_v, sems_in, sems_out,
                          lo, hi, nrows, lane):
  per_block = (128 * nrows) // 128

  if nrows == 64:
    rows_of = [8 * s + (lane >> 1) for s in range(8)]
    colbase = (lane & 1) * _D
  else:
    rows_of = [2 * s + (lane >> 3) for s in range(8)]
    colbase = (lane & 7) * _NLANE

  def start_in(g, b):
    return pltpu.async_copy(tab_t.at[:, pl.ds(g * 128, 128)],
                            blk_v.at[b], sems_in[b])

  def start_out(g, b):
    return pltpu.async_copy(outb_v.at[b],
                            pack_out.at[pl.ds(g * per_block, per_block)],
                            sems_out[b])

  @pl.when(lo < hi)
  def _():
    start_in(lo, 0)

  @pl.when(lo + 1 < hi)
  def _():
    start_in(lo + 1, 1)

  def body(t, carry):
    for b in range(2):
      g = lo + 2 * t + b

      @pl.when(g < hi)
      def _(g=g, b=b):
        pltpu.make_async_copy(tab_t.at[:, pl.ds(g * 128, 128)],
                              blk_v.at[b], sems_in[b]).wait()

        @pl.when(g - 2 >= lo)
        def _(g=g, b=b):
          pltpu.make_async_copy(outb_v.at[b],
                                pack_out.at[pl.ds(g * per_block, per_block)],
                                sems_out[b]).wait()

        for c in range(nrows):
          cols = colbase + c
          for s in range(8):
            v = blk_v[b, c, pl.ds(s * _NLANE, _NLANE)]
            plsc.store_scatter(outb_v.at[b], [rows_of[s], cols], v)
        start_out(g, b)

        @pl.when(g + 2 < hi)
        def _(g=g, b=b):
          start_in(g + 2, b)
    return carry

  n = hi - lo
  lax.fori_loop(0, (n + 1) // 2, body, 0)
  for b in range(2):
    @pl.when(n > b)
    def _(b=b):
      g_last = lo + n - 1 - ((n - 1 - b) % 2)
      pltpu.make_async_copy(outb_v.at[b],
                            pack_out.at[pl.ds(g_last * per_block, per_block)],
                            sems_out[b]).wait()


def _tail_pack(tail_hbm, pack_out, tail_v, outb_v, sem, nrows, base_row, lane):
  pltpu.sync_copy(tail_hbm, tail_v)
  if nrows == 64:
    rows_of = [8 * s + (lane >> 1) for s in range(2)]
    colbase = (lane & 1) * _D
    out_rows = 16
  else:
    rows_of = [2 * s + (lane >> 3) for s in range(2)]
    colbase = (lane & 7) * _NLANE
    out_rows = 4
  for c in range(nrows):
    cols = colbase + c
    for s in range(2):
      v = tail_v[c, pl.ds(s * _NLANE, _NLANE)]
      plsc.store_scatter(outb_v, [rows_of[s], cols], v)
  pltpu.sync_copy(outb_v.at[pl.ds(0, out_rows)],
                  pack_out.at[pl.ds(base_row, out_rows)])


def _sc_body_a(wng_t, win_t, ngm_t, wng_tail, win_tail, ngm_tail,
               wng_pack, win_pack, ngm_pack,
               blk_v, outb_v, blkm_v, outm_v, tail_v, tailo_v,
               tailm_v, tailmo_v,
               sem_i0, sem_i1, sem_o0, sem_o1):
  wid = lax.axis_index("s") * _NC + lax.axis_index("c")
  lane = lax.iota(jnp.int32, _NLANE)
  sems_in = [sem_i0, sem_i1]
  sems_out = [sem_o0, sem_o1]

  lo, hi = _split(wid, _NGB)
  _transpose_pack_phase(wng_t, wng_pack, blk_v, outb_v, sems_in, sems_out,
                        lo, hi, 64, lane)
  lo, hi = _split(wid, _INB)
  _transpose_pack_phase(win_t, win_pack, blk_v, outb_v, sems_in, sems_out,
                        lo, hi, 64, lane)
  _transpose_pack_phase(ngm_t, ngm_pack, blkm_v, outm_v, sems_in, sems_out,
                        lo, hi, 16, lane)

  @pl.when(wid == _NW - 1)
  def _():
    _tail_pack(wng_tail, wng_pack, tail_v, tailo_v, sem_i0, 64,
               _NGB * 64, lane)
    _tail_pack(win_tail, win_pack, tail_v, tailo_v, sem_i0, 64,
               _INB * 64, lane)
    _tail_pack(ngm_tail, ngm_pack, tailm_v, tailmo_v, sem_i0, 16,
               _INB * 16, lane)


def _sc_body_b(word_ids_hbm, w_in_hbm, w_ng_hbm, ngm_hbm, ng_len_hbm,
               out_hbm, idx_v, lens_v, ngmi_v, wini_v, ngflat_v, nghalf_v,
               ngm_rows_v, win_rows_v, ng_rows_v, acc_v,
               sem_len, sem_ngm0, sem_ngm1, sem_win0, sem_win1,
               sem_ng0, sem_ng1):
  wid = lax.axis_index("s") * _NC + lax.axis_index("c")
  base = wid * _BPW
  lane = lax.iota(jnp.int32, _NLANE)

  pltpu.sync_copy(word_ids_hbm.at[pl.ds(base, _BPW)], idx_v)

  def idx_body(g, carry):
    wv = idx_v[pl.ds(g * _NLANE, _NLANE)]
    ngmi_v[pl.ds(g * _NLANE, _NLANE)] = wv >> 3
    wini_v[pl.ds(g * _NLANE, _NLANE)] = wv >> 1
    return carry

  lax.fori_loop(0, _BPW // _NLANE, idx_body, 0)
  cp_len = pltpu.async_copy(ng_len_hbm.at[idx_v], lens_v, sem_len)

  ngm_sems = [sem_ngm0, sem_ngm1]

  def start_ngm(c, buf):
    return pltpu.async_copy(
        ngm_hbm.at[ngmi_v.at[pl.ds(c * _FCH, _FCH)]],
        ngm_rows_v.at[buf], ngm_sems[buf])

  start_ngm(0, 0)

  def flat_chunk(c, carry):
    for b in range(2):
      cc = c * 2 + b
      pltpu.make_async_copy(
          ngm_hbm.at[ngmi_v.at[pl.ds(cc * _FCH, _FCH)]],
          ngm_rows_v.at[b], ngm_sems[b]).wait()

      @pl.when(cc + 1 < _BPW // _FCH)
      def _(cc=cc, b=b):
        start_ngm(cc + 1, 1 - b)

      for g in range(_FCH // _NLANE):
        wv = idx_v[pl.ds(cc * _FCH + g * _NLANE, _NLANE)]
        for wi in range(_NLANE):
          ws = g * _NLANE + wi
          sub = wv[wi] & 7
          idv = ngm_rows_v[b, ws, pl.ds(sub * _NLANE, _NLANE)]
          ngflat_v[pl.ds((cc * _FCH + ws) * _MAX_NG, _MAX_NG)] = idv >> 1
          nghalf_v[pl.ds((cc * _FCH + ws) * _MAX_NG, _MAX_NG)] = idv & 1
    return carry

  lax.fori_loop(0, _BPW // _FCH // 2, flat_chunk, 0)
  cp_len.wait()

  rtab = jnp.full((_NLANE,), 1.0 / (1.0 + _MAX_NG), dtype=jnp.float32)
  for k in range(_MAX_NG - 1):
    rtab = jnp.where(lane == k, jnp.float32(1.0 / (2.0 + k)), rtab)

  win_sems = [sem_win0, sem_win1]
  ng_sems = [sem_ng0, sem_ng1]

  def start_rows(c, buf):
    pltpu.async_copy(
        w_in_hbm.at[wini_v.at[pl.ds(c * _CH, _CH)]],
        win_rows_v.at[buf], win_sems[buf])
    pltpu.async_copy(
        w_ng_hbm.at[ngflat_v.at[pl.ds(c * _CH * _MAX_NG, _CH * _MAX_NG)]],
        ng_rows_v.at[buf], ng_sems[buf])

  start_rows(0, 0)
  start_rows(1, 1)

  def chunk(c2, carry):
    for b in range(2):
      c = c2 * 2 + b
      pltpu.make_async_copy(
          w_in_hbm.at[wini_v.at[pl.ds(c * _CH, _CH)]],
          win_rows_v.at[b], win_sems[b]).wait()
      pltpu.make_async_copy(
          w_ng_hbm.at[ngflat_v.at[pl.ds(c * _CH * _MAX_NG, _CH * _MAX_NG)]],
          ng_rows_v.at[b], ng_sems[b]).wait()

      wv = idx_v[pl.ds(c * _NLANE, _NLANE)]
      lv = lens_v[pl.ds(c * _NLANE, _NLANE)]
      invs = jnp.take(rtab, jnp.clip(lv - 1, 0, _MAX_NG - 1), mode="fill")
      for wi in range(_NLANE):
        hin = (wv[wi] & 1) * _D
        lnc = jnp.minimum(lv[wi], _MAX_NG)
        w = c * _CH + wi
        offv = nghalf_v[pl.ds(w * _MAX_NG, _MAX_NG)] * _D
        accs = tuple(
            win_rows_v[b, wi, pl.ds(hin + d * _NLANE, _NLANE)]
            for d in range(_DV))

        def j_body(j, accs, b=b, wi=wi, offv=offv):
          jv = jnp.full((_NLANE,), j, dtype=jnp.int32)
          off = jnp.take(offv, jv, mode="fill")
          rows = jv + wi * _MAX_NG
          return tuple(
              accs[d] + plsc.load_gather(
                  ng_rows_v.at[b], [rows, off + (d * _NLANE + lane)])
              for d in range(_DV))

        accs = lax.fori_loop(0, lnc, j_body, accs)
        inv = jnp.take(invs, jnp.full((_NLANE,), wi, dtype=jnp.int32),
                       mode="fill")
        arow = c * (_CH // 2) + wi // 2
        aoff = (wi & 1) * _D
        for d in range(_DV):
          acc_v[arow, pl.ds(aoff + d * _NLANE, _NLANE)] = accs[d] * inv

      @pl.when(c + 2 < _NCH)
      def _(c=c, b=b):
        start_rows(c + 2, b)
    return carry

  lax.fori_loop(0, _NCH // 2, chunk, 0)
  pltpu.sync_copy(acc_v, out_hbm.at[pl.ds(wid * (_BPW // 2), _BPW // 2)])


@jax.jit
def kernel(word_ids, W_in, W_ng, ng_matrix, ng_lengths):
  mesh = plsc.VectorSubcoreMesh(core_axis_name="c", subcore_axis_name="s")
  run_a = functools.partial(
      pl.kernel,
      out_type=(
          jax.ShapeDtypeStruct((250000, 2 * _D), jnp.float32),
          jax.ShapeDtypeStruct((_VOCAB // 2, 2 * _D), jnp.float32),
          jax.ShapeDtypeStruct((_VOCAB // 8, 8 * _MAX_NG), jnp.int32),
      ),
      mesh=mesh,
      compiler_params=pltpu.CompilerParams(needs_layout_passes=False),
      scratch_types=[
          pltpu.VMEM((2, 64, 128), jnp.float32),
          pltpu.VMEM((2, 64, 128), jnp.float32),
          pltpu.VMEM((2, 16, 128), jnp.int32),
          pltpu.VMEM((2, 16, 128), jnp.int32),
          pltpu.VMEM((64, 32), jnp.float32),
          pltpu.VMEM((16, 128), jnp.float32),
          pltpu.VMEM((16, 32), jnp.int32),
          pltpu.VMEM((16, 128), jnp.int32),
          pltpu.SemaphoreType.DMA,
          pltpu.SemaphoreType.DMA,
          pltpu.SemaphoreType.DMA,
          pltpu.SemaphoreType.DMA,
      ],
  )(_sc_body_a)
  run_b = functools.partial(
      pl.kernel,
      out_type=jax.ShapeDtypeStruct((_B // 2, 2 * _D), jnp.float32),
      mesh=mesh,
      compiler_params=pltpu.CompilerParams(needs_layout_passes=False),
      scratch_types=[
          pltpu.VMEM((_BPW,), jnp.int32),
          pltpu.VMEM((_BPW,), jnp.int32),
          pltpu.VMEM((_BPW,), jnp.int32),
          pltpu.VMEM((_BPW,), jnp.int32),
          pltpu.VMEM((_BPW * _MAX_NG,), jnp.int32),
          pltpu.VMEM((_BPW * _MAX_NG,), jnp.int32),
          pltpu.VMEM((2, _FCH, 2 * _D), jnp.int32),
          pltpu.VMEM((2, _CH, 2 * _D), jnp.float32),
          pltpu.VMEM((2, _CH * _MAX_NG, 2 * _D), jnp.float32),
          pltpu.VMEM((_BPW // 2, 2 * _D), jnp.float32),
          pltpu.SemaphoreType.DMA,
          pltpu.SemaphoreType.DMA,
          pltpu.SemaphoreType.DMA,
          pltpu.SemaphoreType.DMA,
          pltpu.SemaphoreType.DMA,
          pltpu.SemaphoreType.DMA,
          pltpu.SemaphoreType.DMA,
      ],
  )(_sc_body_b)

  wng_pack, win_pack, ngm_pack = run_a(
      W_ng.T, W_in.T, ng_matrix.T,
      W_ng[_NGB * 128:].T, W_in[_INB * 128:].T, ng_matrix[_INB * 128:].T)
  out2 = run_b(word_ids, win_pack, wng_pack, ngm_pack, ng_lengths)
  return out2.reshape(_B, _D)
